# baseline (speedup 1.0000x reference)
import functools

import jax
import jax.numpy as jnp
from jax import lax
from jax.experimental import pallas as pl
from jax.experimental.pallas import tpu as pltpu
from jax.experimental.pallas import tpu_sc as plsc

NUM_ROWS = 1_000_000
D = 64
BATCH = 16384
FEATS = 26
NC, NS = 2, 16
NW = NC * NS
BATCH_W = BATCH // NW
GB = 128
SPANS = BATCH_W // GB
NCHUNK = FEATS * SPANS
L = 16


def _sc_gather_t(x_t, table):
    mesh = plsc.VectorSubcoreMesh(core_axis_name="c", subcore_axis_name="s")

    @functools.partial(
        pl.kernel,
        out_type=jax.ShapeDtypeStruct((FEATS, D, BATCH), jnp.float32),
        mesh=mesh,
        scratch_types=[
            pltpu.VMEM((FEATS, BATCH_W), jnp.int32),
            pltpu.VMEM((2, GB, D), jnp.float32),
            pltpu.VMEM((2, D, GB), jnp.float32),
            [pltpu.SemaphoreType.DMA] * 2,
            [pltpu.SemaphoreType.DMA] * 2,
        ],
        compiler_params=pltpu.CompilerParams(
            use_tc_tiling_on_sc=False, needs_layout_passes=False
        ),
    )
    def k(xt_hbm, table_hbm, out_hbm, idx_v, gbuf, tbuf, gsems, osems):
        wid = lax.axis_index("s") * NC + lax.axis_index("c")
        wb0 = wid * BATCH_W
        pltpu.sync_copy(xt_hbm.at[:, pl.ds(wb0, BATCH_W)], idx_v)

        def fire(s, b):
            f = s // SPANS
            k0 = (s % SPANS) * GB
            pltpu.async_copy(
                table_hbm.at[idx_v.at[f, pl.ds(k0, GB)]],
                gbuf.at[b],
                gsems[b],
            )

        def wait_gather(b):
            pltpu.make_async_copy(
                table_hbm.at[pl.ds(0, GB)], gbuf.at[b], gsems[b]
            ).wait()

        def transpose(b):
            for blk in range(GB // L):
                row_idx = lax.iota(jnp.int32, L) + blk * L
                for d in range(D):
                    col_idx = jnp.full((L,), d, jnp.int32)
                    vals = plsc.load_gather(gbuf.at[b], [row_idx, col_idx])
                    tbuf[b, d, pl.ds(blk * L, L)] = vals

        def put(s, b):
            f = s // SPANS
            k0 = (s % SPANS) * GB
            pltpu.async_copy(
                tbuf.at[b],
                out_hbm.at[f, :, pl.ds(wb0 + k0, GB)],
                osems[b],
            )

        def drain_out(b):
            pltpu.make_async_copy(
                tbuf.at[b], out_hbm.at[0, :, pl.ds(0, GB)], osems[b]
            ).wait()

        fire(0, 0)
        fire(1, 1)

        def body(t, carry):
            for b in range(2):
                s = t * 2 + b
                wait_gather(b)
                transpose(b)
                fire(s + 2, b)
                put(s, b)
                drain_out(b)
            return carry

        lax.fori_loop(0, NCHUNK // 2 - 1, body, 0)

        for b in range(2):
            s = NCHUNK - 2 + b
            wait_gather(b)
            transpose(b)
            put(s, b)
            drain_out(b)

    return k(x_t, table)


def kernel(x, table):
    out_t = _sc_gather_t(x.T, table)
    return jnp.transpose(out_t, (2, 0, 1))


# speedup vs baseline: 1.1790x; 1.1790x over previous
import functools

import jax
import jax.numpy as jnp
from jax import lax
from jax.experimental import pallas as pl
from jax.experimental.pallas import tpu as pltpu
from jax.experimental.pallas import tpu_sc as plsc

NUM_ROWS = 1_000_000
D = 64
BATCH = 16384
FEATS = 26
NC, NS = 2, 16
NW = NC * NS
BATCH_W = BATCH // NW
GB = 128
SPANS = BATCH_W // GB
NCHUNK = FEATS * SPANS
L = 16
JU = 8


def _sc_gather_t(x_t, table):
    mesh = plsc.VectorSubcoreMesh(core_axis_name="c", subcore_axis_name="s")

    @functools.partial(
        pl.kernel,
        out_type=jax.ShapeDtypeStruct((FEATS, D, BATCH), jnp.float32),
        mesh=mesh,
        scratch_types=[
            pltpu.VMEM((FEATS, BATCH_W), jnp.int32),
            pltpu.VMEM((2, GB, D), jnp.float32),
            pltpu.VMEM((2, D, GB), jnp.float32),
            [pltpu.SemaphoreType.DMA] * 2,
            [pltpu.SemaphoreType.DMA] * 2,
        ],
        compiler_params=pltpu.CompilerParams(
            use_tc_tiling_on_sc=False, needs_layout_passes=False
        ),
    )
    def k(xt_hbm, table_hbm, out_hbm, idx_v, gbuf, tbuf, gsems, osems):
        wid = lax.axis_index("s") * NC + lax.axis_index("c")
        wb0 = wid * BATCH_W
        pltpu.sync_copy(xt_hbm.at[:, pl.ds(wb0, BATCH_W)], idx_v)

        def fire(s, b):
            f = s // SPANS
            k0 = (s % SPANS) * GB
            pltpu.async_copy(
                table_hbm.at[idx_v.at[f, pl.ds(k0, GB)]],
                gbuf.at[b],
                gsems[b],
            )

        def wait_gather(b):
            pltpu.make_async_copy(
                table_hbm.at[pl.ds(0, GB)], gbuf.at[b], gsems[b]
            ).wait()

        def transpose(b):
            d_iota = [
                lax.iota(jnp.int32, L) + d0 for d0 in range(0, D, L)
            ]
            col1 = jnp.full((L,), 1, jnp.int32)

            def jbody(t, carry):
                for ju in range(JU):
                    j = t * JU + ju
                    jvec = col1 * j
                    for di, d0 in enumerate(range(0, D, L)):
                        vals = gbuf[b, j, pl.ds(d0, L)]
                        plsc.store_scatter(
                            tbuf.at[b], [d_iota[di], jvec], vals
                        )
                return carry

            lax.fori_loop(0, GB // JU, jbody, 0)

        def put(s, b):
            f = s // SPANS
            k0 = (s % SPANS) * GB
            pltpu.async_copy(
                tbuf.at[b],
                out_hbm.at[f, :, pl.ds(wb0 + k0, GB)],
                osems[b],
            )

        def drain_out(b):
            pltpu.make_async_copy(
                tbuf.at[b], out_hbm.at[0, :, pl.ds(0, GB)], osems[b]
            ).wait()

        fire(0, 0)
        fire(1, 1)

        def body(t, carry):
            for b in range(2):
                s = t * 2 + b
                wait_gather(b)
                transpose(b)
                fire(s + 2, b)
                put(s, b)
                drain_out(b)
            return carry

        lax.fori_loop(0, NCHUNK // 2 - 1, body, 0)

        for b in range(2):
            s = NCHUNK - 2 + b
            wait_gather(b)
            transpose(b)
            put(s, b)
            drain_out(b)

    return k(x_t, table)


def kernel(x, table):
    out_t = _sc_gather_t(x.T, table)
    return jnp.transpose(out_t, (2, 0, 1))


# speedup vs baseline: 1.5890x; 1.3477x over previous
import functools

import jax
import jax.numpy as jnp
from jax import lax
from jax.experimental import pallas as pl
from jax.experimental.pallas import tpu as pltpu
from jax.experimental.pallas import tpu_sc as plsc

NUM_ROWS = 1_000_000
D = 64
BATCH = 16384
FEATS = 26
NC, NS = 2, 16
NW = NC * NS
BATCH_W = BATCH // NW
GB = 128
SPANS = BATCH_W // GB
NCHUNK = FEATS * SPANS
L = 16
JU = 8
GBP = GB + 1


def _sc_gather_t(x_t, table):
    mesh = plsc.VectorSubcoreMesh(core_axis_name="c", subcore_axis_name="s")

    @functools.partial(
        pl.kernel,
        out_type=jax.ShapeDtypeStruct((FEATS, D, BATCH), jnp.float32),
        mesh=mesh,
        scratch_types=[
            pltpu.VMEM((FEATS, BATCH_W), jnp.int32),
            pltpu.VMEM((2, GB, D), jnp.float32),
            pltpu.VMEM((2, D, GBP), jnp.float32),
            [pltpu.SemaphoreType.DMA] * 2,
            [pltpu.SemaphoreType.DMA] * 2,
        ],
        compiler_params=pltpu.CompilerParams(
            use_tc_tiling_on_sc=False, needs_layout_passes=False
        ),
    )
    def k(xt_hbm, table_hbm, out_hbm, idx_v, gbuf, tbuf, gsems, osems):
        wid = lax.axis_index("s") * NC + lax.axis_index("c")
        wb0 = wid * BATCH_W
        pltpu.sync_copy(xt_hbm.at[:, pl.ds(wb0, BATCH_W)], idx_v)

        def fire(s, b):
            f = s // SPANS
            k0 = (s % SPANS) * GB
            pltpu.async_copy(
                table_hbm.at[idx_v.at[f, pl.ds(k0, GB)]],
                gbuf.at[b],
                gsems[b],
            )

        def wait_gather(b):
            pltpu.make_async_copy(
                table_hbm.at[pl.ds(0, GB)], gbuf.at[b], gsems[b]
            ).wait()

        def transpose(b):
            d_iota = [
                lax.iota(jnp.int32, L) + d0 for d0 in range(0, D, L)
            ]
            col1 = jnp.full((L,), 1, jnp.int32)

            def jbody(t, carry):
                for ju in range(JU):
                    j = t * JU + ju
                    jvec = col1 * j
                    for di, d0 in enumerate(range(0, D, L)):
                        vals = gbuf[b, j, pl.ds(d0, L)]
                        plsc.store_scatter(
                            tbuf.at[b], [d_iota[di], jvec], vals
                        )
                return carry

            lax.fori_loop(0, GB // JU, jbody, 0)

        def put(s, b):
            f = s // SPANS
            k0 = (s % SPANS) * GB
            pltpu.async_copy(
                tbuf.at[b, :, pl.ds(0, GB)],
                out_hbm.at[f, :, pl.ds(wb0 + k0, GB)],
                osems[b],
            )

        def drain_out(b):
            pltpu.make_async_copy(
                tbuf.at[b, :, pl.ds(0, GB)],
                out_hbm.at[0, :, pl.ds(0, GB)],
                osems[b],
            ).wait()

        fire(0, 0)
        fire(1, 1)

        def body(t, carry):
            for b in range(2):
                s = t * 2 + b
                wait_gather(b)
                transpose(b)
                fire(s + 2, b)
                put(s, b)
                drain_out(b)
            return carry

        lax.fori_loop(0, NCHUNK // 2 - 1, body, 0)

        for b in range(2):
            s = NCHUNK - 2 + b
            wait_gather(b)
            transpose(b)
            put(s, b)
            drain_out(b)

    return k(x_t, table)


def kernel(x, table):
    out_t = _sc_gather_t(x.T, table)
    return jnp.transpose(out_t, (2, 0, 1))
